# baseline (speedup 1.0000x reference)
import functools

import jax
import jax.numpy as jnp
from jax import lax
from jax.experimental import pallas as pl
from jax.experimental.pallas import tpu as pltpu
from jax.experimental.pallas import tpu_sc as plsc

B = 512
H = 28
W = 28
N = W * H * 3
K = int(0.75 * N)
N_EXTRA = N - K
NPAD = 2432
KPAD = 1792
LANES = 16
NCHUNK = N // LANES
NC, NS = 2, 16
NW = NC * NS
ROWS_PER_W = B // NW
SENTINEL = 1e30

@functools.cache
def _sc_mesh():
    return plsc.VectorSubcoreMesh(
        core_axis_name="c", subcore_axis_name="s",
        num_cores=NC, num_subcores=NS)


def _sc_interp_body(img_hbm, ux_hbm, uy_hbm, out_hbm, img_v, ux_v, uy_v, out_v):
    wid = lax.axis_index("s") * NC + lax.axis_index("c")
    base = wid * ROWS_PER_W

    def row_body(r, carry):
        b = base + r
        pltpu.sync_copy(img_hbm.at[b], img_v)
        pltpu.sync_copy(ux_hbm.at[b], ux_v)
        pltpu.sync_copy(uy_hbm.at[b], uy_v)

        def chunk_body(i, c2):
            s = i * LANES
            u0 = ux_v[pl.ds(s, LANES)]
            u1 = uy_v[pl.ds(s, LANES)]
            i0 = u0.astype(jnp.int32)
            i1 = u1.astype(jnp.int32)
            f0 = i0.astype(jnp.float32)
            f1 = i1.astype(jnp.float32)
            m0 = u0 - f0
            m1 = u1 - f1
            c0 = i0 + (u0 > f0).astype(jnp.int32)
            c1 = i1 + (u1 > f1).astype(jnp.int32)
            p1 = plsc.load_gather(img_v, [i0 * W + i1])
            p2 = plsc.load_gather(img_v, [c0 * W + i1])
            p3 = plsc.load_gather(img_v, [i0 * W + c1])
            p4 = plsc.load_gather(img_v, [c0 * W + c1])
            a = p1 * (1.0 - m0) + p2 * m0
            bq = p3 * (1.0 - m0) + p4 * m0
            v = a * (1.0 - m1) + bq * m1
            out_v[pl.ds(s, LANES)] = jnp.abs(v)
            return c2

        lax.fori_loop(0, NCHUNK, chunk_body, 0)
        for t in range(NCHUNK, NPAD // LANES):
            out_v[pl.ds(t * LANES, LANES)] = jnp.full((LANES,), SENTINEL,
                                                      jnp.float32)
        pltpu.sync_copy(out_v, out_hbm.at[b])
        return carry

    lax.fori_loop(0, ROWS_PER_W, row_body, 0)


@functools.cache
def _sc_interp():
    return pl.kernel(
        _sc_interp_body,
        out_type=jax.ShapeDtypeStruct((B, NPAD), jnp.float32),
        mesh=_sc_mesh(),
        compiler_params=pltpu.CompilerParams(use_tc_tiling_on_sc=False,
                                            needs_layout_passes=False),
        scratch_types=[
            pltpu.VMEM((H * W,), jnp.float32),
            pltpu.VMEM((N,), jnp.float32),
            pltpu.VMEM((N,), jnp.float32),
            pltpu.VMEM((NPAD,), jnp.float32),
        ],
    )


def _tc_rank_body(u_ref, ut_ref, rank_ref):
    nt = NPAD // 128
    sub = lax.broadcasted_iota(jnp.int32, (128, 128), 0)
    lane = lax.broadcasted_iota(jnp.int32, (128, 128), 1)

    def rank_chunk(ic, carry):
        ui = u_ref[0, :, pl.ds(ic * 128, 128)]

        def acc_body(jb, acc):
            ujt = ut_ref[0, pl.ds(jb * 128, 128), :]
            lt = ujt < ui
            eq = ujt == ui
            jlt = (jb * 128 + sub) < (ic * 128 + lane)
            cnt = jnp.where(lt | (eq & jlt), 1.0, 0.0)
            return acc + cnt

        acc = lax.fori_loop(0, nt, acc_body, jnp.zeros((128, 128), jnp.float32))
        rank = jnp.sum(acc, axis=0, keepdims=True).astype(jnp.int32)
        rank_ref[0, :, pl.ds(ic * 128, 128)] = rank
        return carry

    lax.fori_loop(0, nt, rank_chunk, 0)


def _tc_ranks(u):
    u3 = u.reshape(B, 1, NPAD)
    ut = u.reshape(B, NPAD, 1)
    out = pl.pallas_call(
        _tc_rank_body,
        grid=(B,),
        in_specs=[
            pl.BlockSpec((1, 1, NPAD), lambda b: (b, 0, 0)),
            pl.BlockSpec((1, NPAD, 1), lambda b: (b, 0, 0)),
        ],
        out_specs=pl.BlockSpec((1, 1, NPAD), lambda b: (b, 0, 0)),
        out_shape=jax.ShapeDtypeStruct((B, 1, NPAD), jnp.int32),
    )(u3, ut)
    return out.reshape(B, NPAD)


def _sc_scatter_body(rank_hbm, cx_hbm, cy_hbm, ox_hbm, oy_hbm,
                     rank_v, cx_v, cy_v, ox_v, oy_v):
    wid = lax.axis_index("s") * NC + lax.axis_index("c")
    base = wid * ROWS_PER_W

    def row_body(r, carry):
        b = base + r
        pltpu.sync_copy(rank_hbm.at[b], rank_v)
        pltpu.sync_copy(cx_hbm.at[b], cx_v)
        pltpu.sync_copy(cy_hbm.at[b], cy_v)

        def chunk_body(i, c2):
            s = i * LANES
            rk = rank_v[pl.ds(s, LANES)]
            msk = rk < K
            plsc.store_scatter(ox_v, [rk], cx_v[pl.ds(s, LANES)], mask=msk)
            plsc.store_scatter(oy_v, [rk], cy_v[pl.ds(s, LANES)], mask=msk)
            return c2

        lax.fori_loop(0, NCHUNK, chunk_body, 0)
        pltpu.sync_copy(ox_v, ox_hbm.at[b])
        pltpu.sync_copy(oy_v, oy_hbm.at[b])
        return carry

    lax.fori_loop(0, ROWS_PER_W, row_body, 0)


@functools.cache
def _sc_scatter():
    return pl.kernel(
        _sc_scatter_body,
        out_type=(jax.ShapeDtypeStruct((B, KPAD), jnp.float32),
                  jax.ShapeDtypeStruct((B, KPAD), jnp.float32)),
        mesh=_sc_mesh(),
        compiler_params=pltpu.CompilerParams(use_tc_tiling_on_sc=False,
                                            needs_layout_passes=False),
        scratch_types=[
            pltpu.VMEM((NPAD,), jnp.int32),
            pltpu.VMEM((N,), jnp.float32),
            pltpu.VMEM((N,), jnp.float32),
            pltpu.VMEM((KPAD,), jnp.float32),
            pltpu.VMEM((KPAD,), jnp.float32),
        ],
    )


def kernel(mask_coarse_logits):
    img = mask_coarse_logits[:, :, :, 0].reshape(B, H * W)
    coords = jax.random.uniform(jax.random.key(1), (B, N, 2),
                                dtype=jnp.float32)
    unnorm = coords * jnp.array([H - 1, W - 1], dtype=jnp.float32)
    ux = unnorm[:, :, 0]
    uy = unnorm[:, :, 1]
    u_abs = _sc_interp()(img, ux, uy)
    ranks = _tc_ranks(u_abs)
    cx = coords[:, :, 0]
    cy = coords[:, :, 1]
    ox, oy = _sc_scatter()(ranks, cx, cy)
    top = jnp.stack([ox[:, :K], oy[:, :K]], axis=-1)
    extra = jax.random.uniform(jax.random.key(2), (B, N_EXTRA, 2),
                               dtype=jnp.float32)
    return jnp.concatenate([top, extra], axis=1)


# speedup vs baseline: 5.1400x; 5.1400x over previous
import functools

import jax
import jax.numpy as jnp
from jax import lax
from jax.experimental import pallas as pl
from jax.experimental.pallas import tpu as pltpu
from jax.experimental.pallas import tpu_sc as plsc

B = 512
H = 28
W = 28
N = W * H * 3
K = int(0.75 * N)
N_EXTRA = N - K
NPAD = 2432
KPAD = 1792
LANES = 16
NCHUNK = N // LANES
NC, NS = 2, 16
NW = NC * NS
ROWS_PER_W = B // NW
SENTINEL = 1e30

@functools.cache
def _sc_mesh():
    return plsc.VectorSubcoreMesh(
        core_axis_name="c", subcore_axis_name="s",
        num_cores=NC, num_subcores=NS)


def _sc_interp_body(img_hbm, ux_hbm, uy_hbm, out_hbm, img_v, ux_v, uy_v, out_v):
    wid = lax.axis_index("s") * NC + lax.axis_index("c")
    base = wid * ROWS_PER_W

    def row_body(r, carry):
        b = base + r
        pltpu.sync_copy(img_hbm.at[b], img_v)
        pltpu.sync_copy(ux_hbm.at[b], ux_v)
        pltpu.sync_copy(uy_hbm.at[b], uy_v)

        def chunk_body(i, c2):
            s = i * LANES
            u0 = ux_v[pl.ds(s, LANES)]
            u1 = uy_v[pl.ds(s, LANES)]
            i0 = u0.astype(jnp.int32)
            i1 = u1.astype(jnp.int32)
            f0 = i0.astype(jnp.float32)
            f1 = i1.astype(jnp.float32)
            m0 = u0 - f0
            m1 = u1 - f1
            c0 = i0 + (u0 > f0).astype(jnp.int32)
            c1 = i1 + (u1 > f1).astype(jnp.int32)
            p1 = plsc.load_gather(img_v, [i0 * W + i1])
            p2 = plsc.load_gather(img_v, [c0 * W + i1])
            p3 = plsc.load_gather(img_v, [i0 * W + c1])
            p4 = plsc.load_gather(img_v, [c0 * W + c1])
            a = p1 * (1.0 - m0) + p2 * m0
            bq = p3 * (1.0 - m0) + p4 * m0
            v = a * (1.0 - m1) + bq * m1
            out_v[pl.ds(s, LANES)] = jnp.abs(v)
            return c2

        lax.fori_loop(0, NCHUNK, chunk_body, 0)
        for t in range(NCHUNK, NPAD // LANES):
            out_v[pl.ds(t * LANES, LANES)] = jnp.full((LANES,), SENTINEL,
                                                      jnp.float32)
        pltpu.sync_copy(out_v, out_hbm.at[b])
        return carry

    lax.fori_loop(0, ROWS_PER_W, row_body, 0)


@functools.cache
def _sc_interp():
    return pl.kernel(
        _sc_interp_body,
        out_type=jax.ShapeDtypeStruct((B, NPAD), jnp.float32),
        mesh=_sc_mesh(),
        compiler_params=pltpu.CompilerParams(use_tc_tiling_on_sc=False,
                                            needs_layout_passes=False),
        scratch_types=[
            pltpu.VMEM((H * W,), jnp.float32),
            pltpu.VMEM((N,), jnp.float32),
            pltpu.VMEM((N,), jnp.float32),
            pltpu.VMEM((NPAD,), jnp.float32),
        ],
    )


def _tc_rank_body(u_ref, ut_ref, rank_ref, bc_ref):
    nt = NPAD // 128
    sub = lax.broadcasted_iota(jnp.int32, (128, 128), 0)
    lane = lax.broadcasted_iota(jnp.int32, (128, 128), 1)
    tri = sub < lane

    for jb in range(nt):
        ujt = ut_ref[0, pl.ds(jb * 128, 128), :]
        bc_ref[pl.ds(jb * 128, 128), :] = jnp.broadcast_to(ujt, (128, 128))

    def rank_chunk(ic, carry):
        ui = u_ref[0, :, pl.ds(ic * 128, 128)]

        def le_body(jb, acc):
            t = bc_ref[pl.ds(jb * 128, 128), :]
            return acc + jnp.where(t <= ui, 1.0, 0.0)

        def lt_body(jb, acc):
            t = bc_ref[pl.ds(jb * 128, 128), :]
            return acc + jnp.where(t < ui, 1.0, 0.0)

        acc = lax.fori_loop(0, ic, le_body,
                            jnp.zeros((128, 128), jnp.float32))
        td = bc_ref[pl.ds(ic * 128, 128), :]
        acc = acc + jnp.where((td < ui) | ((td <= ui) & tri), 1.0, 0.0)
        acc = lax.fori_loop(ic + 1, nt, lt_body, acc)
        rank = jnp.sum(acc, axis=0, keepdims=True).astype(jnp.int32)
        rank_ref[0, :, pl.ds(ic * 128, 128)] = rank
        return carry

    lax.fori_loop(0, nt, rank_chunk, 0)


def _tc_ranks(u):
    u3 = u.reshape(B, 1, NPAD)
    ut = u.reshape(B, NPAD, 1)
    out = pl.pallas_call(
        _tc_rank_body,
        grid=(B,),
        in_specs=[
            pl.BlockSpec((1, 1, NPAD), lambda b: (b, 0, 0)),
            pl.BlockSpec((1, NPAD, 1), lambda b: (b, 0, 0)),
        ],
        out_specs=pl.BlockSpec((1, 1, NPAD), lambda b: (b, 0, 0)),
        out_shape=jax.ShapeDtypeStruct((B, 1, NPAD), jnp.int32),
        scratch_shapes=[pltpu.VMEM((NPAD, 128), jnp.float32)],
    )(u3, ut)
    return out.reshape(B, NPAD)


def _sc_scatter_body(rank_hbm, cx_hbm, cy_hbm, ox_hbm, oy_hbm,
                     rank_v, cx_v, cy_v, ox_v, oy_v):
    wid = lax.axis_index("s") * NC + lax.axis_index("c")
    base = wid * ROWS_PER_W

    def row_body(r, carry):
        b = base + r
        pltpu.sync_copy(rank_hbm.at[b], rank_v)
        pltpu.sync_copy(cx_hbm.at[b], cx_v)
        pltpu.sync_copy(cy_hbm.at[b], cy_v)

        def chunk_body(i, c2):
            s = i * LANES
            rk = rank_v[pl.ds(s, LANES)]
            msk = rk < K
            plsc.store_scatter(ox_v, [rk], cx_v[pl.ds(s, LANES)], mask=msk)
            plsc.store_scatter(oy_v, [rk], cy_v[pl.ds(s, LANES)], mask=msk)
            return c2

        lax.fori_loop(0, NCHUNK, chunk_body, 0)
        pltpu.sync_copy(ox_v, ox_hbm.at[b])
        pltpu.sync_copy(oy_v, oy_hbm.at[b])
        return carry

    lax.fori_loop(0, ROWS_PER_W, row_body, 0)


@functools.cache
def _sc_scatter():
    return pl.kernel(
        _sc_scatter_body,
        out_type=(jax.ShapeDtypeStruct((B, KPAD), jnp.float32),
                  jax.ShapeDtypeStruct((B, KPAD), jnp.float32)),
        mesh=_sc_mesh(),
        compiler_params=pltpu.CompilerParams(use_tc_tiling_on_sc=False,
                                            needs_layout_passes=False),
        scratch_types=[
            pltpu.VMEM((NPAD,), jnp.int32),
            pltpu.VMEM((N,), jnp.float32),
            pltpu.VMEM((N,), jnp.float32),
            pltpu.VMEM((KPAD,), jnp.float32),
            pltpu.VMEM((KPAD,), jnp.float32),
        ],
    )


def kernel(mask_coarse_logits):
    img = mask_coarse_logits[:, :, :, 0].reshape(B, H * W)
    coords = jax.random.uniform(jax.random.key(1), (B, N, 2),
                                dtype=jnp.float32)
    unnorm = coords * jnp.array([H - 1, W - 1], dtype=jnp.float32)
    ux = unnorm[:, :, 0]
    uy = unnorm[:, :, 1]
    u_abs = _sc_interp()(img, ux, uy)
    ranks = _tc_ranks(u_abs)
    cx = coords[:, :, 0]
    cy = coords[:, :, 1]
    ox, oy = _sc_scatter()(ranks, cx, cy)
    top = jnp.stack([ox[:, :K], oy[:, :K]], axis=-1)
    extra = jax.random.uniform(jax.random.key(2), (B, N_EXTRA, 2),
                               dtype=jnp.float32)
    return jnp.concatenate([top, extra], axis=1)


# speedup vs baseline: 8.1248x; 1.5807x over previous
import functools

import jax
import jax.numpy as jnp
from jax import lax
from jax.experimental import pallas as pl
from jax.experimental.pallas import tpu as pltpu
from jax.experimental.pallas import tpu_sc as plsc

B = 512
H = 28
W = 28
N = W * H * 3
K = int(0.75 * N)
N_EXTRA = N - K
NPAD = 2432
KPAD = 1792
LANES = 16
NCHUNK = N // LANES
NC, NS = 2, 16
NW = NC * NS
ROWS_PER_W = B // NW
SENTINEL = 1e30

@functools.cache
def _sc_mesh():
    return plsc.VectorSubcoreMesh(
        core_axis_name="c", subcore_axis_name="s",
        num_cores=NC, num_subcores=NS)


def _sc_interp_body(img_hbm, ux_hbm, uy_hbm, out_hbm, img_v, ux_v, uy_v, out_v):
    wid = lax.axis_index("s") * NC + lax.axis_index("c")
    base = wid * ROWS_PER_W

    def row_body(r, carry):
        b = base + r
        pltpu.sync_copy(img_hbm.at[b], img_v)
        pltpu.sync_copy(ux_hbm.at[b], ux_v)
        pltpu.sync_copy(uy_hbm.at[b], uy_v)

        def chunk_body(i, c2):
            s = i * LANES
            u0 = ux_v[pl.ds(s, LANES)]
            u1 = uy_v[pl.ds(s, LANES)]
            i0 = u0.astype(jnp.int32)
            i1 = u1.astype(jnp.int32)
            f0 = i0.astype(jnp.float32)
            f1 = i1.astype(jnp.float32)
            m0 = u0 - f0
            m1 = u1 - f1
            c0 = i0 + (u0 > f0).astype(jnp.int32)
            c1 = i1 + (u1 > f1).astype(jnp.int32)
            p1 = plsc.load_gather(img_v, [i0 * W + i1])
            p2 = plsc.load_gather(img_v, [c0 * W + i1])
            p3 = plsc.load_gather(img_v, [i0 * W + c1])
            p4 = plsc.load_gather(img_v, [c0 * W + c1])
            a = p1 * (1.0 - m0) + p2 * m0
            bq = p3 * (1.0 - m0) + p4 * m0
            v = a * (1.0 - m1) + bq * m1
            out_v[pl.ds(s, LANES)] = jnp.abs(v)
            return c2

        lax.fori_loop(0, NCHUNK, chunk_body, 0)
        for t in range(NCHUNK, NPAD // LANES):
            out_v[pl.ds(t * LANES, LANES)] = jnp.full((LANES,), SENTINEL,
                                                      jnp.float32)
        pltpu.sync_copy(out_v, out_hbm.at[b])
        return carry

    lax.fori_loop(0, ROWS_PER_W, row_body, 0)


@functools.cache
def _sc_interp():
    return pl.kernel(
        _sc_interp_body,
        out_type=jax.ShapeDtypeStruct((B, NPAD), jnp.float32),
        mesh=_sc_mesh(),
        compiler_params=pltpu.CompilerParams(use_tc_tiling_on_sc=False,
                                            needs_layout_passes=False),
        scratch_types=[
            pltpu.VMEM((H * W,), jnp.float32),
            pltpu.VMEM((N,), jnp.float32),
            pltpu.VMEM((N,), jnp.float32),
            pltpu.VMEM((NPAD,), jnp.float32),
        ],
    )


def _tc_rank_body(u_ref, ut_ref, rank_ref, bc_ref):
    nt = NPAD // 128
    sub = lax.broadcasted_iota(jnp.int32, (128, 128), 0)
    lane = lax.broadcasted_iota(jnp.int32, (128, 128), 1)
    tri = sub < lane

    for jb in range(nt):
        ujt = ut_ref[0, pl.ds(jb * 128, 128), :]
        bc_ref[pl.ds(jb * 128, 128), :] = jnp.broadcast_to(ujt, (128, 128))

    for ic in range(nt):
        ui = u_ref[0, :, pl.ds(ic * 128, 128)]
        acc = jnp.zeros((128, 128), jnp.float32)
        for jb in range(nt):
            t = bc_ref[pl.ds(jb * 128, 128), :]
            if jb < ic:
                acc = acc + jnp.where(t <= ui, 1.0, 0.0)
            elif jb == ic:
                acc = acc + jnp.where((t < ui) | ((t <= ui) & tri), 1.0, 0.0)
            else:
                acc = acc + jnp.where(t < ui, 1.0, 0.0)
        rank = jnp.sum(acc, axis=0, keepdims=True).astype(jnp.int32)
        rank_ref[0, :, pl.ds(ic * 128, 128)] = rank


def _tc_ranks(u):
    u3 = u.reshape(B, 1, NPAD)
    ut = u.reshape(B, NPAD, 1)
    out = pl.pallas_call(
        _tc_rank_body,
        grid=(B,),
        in_specs=[
            pl.BlockSpec((1, 1, NPAD), lambda b: (b, 0, 0)),
            pl.BlockSpec((1, NPAD, 1), lambda b: (b, 0, 0)),
        ],
        out_specs=pl.BlockSpec((1, 1, NPAD), lambda b: (b, 0, 0)),
        out_shape=jax.ShapeDtypeStruct((B, 1, NPAD), jnp.int32),
        scratch_shapes=[pltpu.VMEM((NPAD, 128), jnp.float32)],
    )(u3, ut)
    return out.reshape(B, NPAD)


def _sc_scatter_body(rank_hbm, cx_hbm, cy_hbm, ox_hbm, oy_hbm,
                     rank_v, cx_v, cy_v, ox_v, oy_v):
    wid = lax.axis_index("s") * NC + lax.axis_index("c")
    base = wid * ROWS_PER_W

    def row_body(r, carry):
        b = base + r
        pltpu.sync_copy(rank_hbm.at[b], rank_v)
        pltpu.sync_copy(cx_hbm.at[b], cx_v)
        pltpu.sync_copy(cy_hbm.at[b], cy_v)

        def chunk_body(i, c2):
            s = i * LANES
            rk = rank_v[pl.ds(s, LANES)]
            msk = rk < K
            plsc.store_scatter(ox_v, [rk], cx_v[pl.ds(s, LANES)], mask=msk)
            plsc.store_scatter(oy_v, [rk], cy_v[pl.ds(s, LANES)], mask=msk)
            return c2

        lax.fori_loop(0, NCHUNK, chunk_body, 0)
        pltpu.sync_copy(ox_v, ox_hbm.at[b])
        pltpu.sync_copy(oy_v, oy_hbm.at[b])
        return carry

    lax.fori_loop(0, ROWS_PER_W, row_body, 0)


@functools.cache
def _sc_scatter():
    return pl.kernel(
        _sc_scatter_body,
        out_type=(jax.ShapeDtypeStruct((B, KPAD), jnp.float32),
                  jax.ShapeDtypeStruct((B, KPAD), jnp.float32)),
        mesh=_sc_mesh(),
        compiler_params=pltpu.CompilerParams(use_tc_tiling_on_sc=False,
                                            needs_layout_passes=False),
        scratch_types=[
            pltpu.VMEM((NPAD,), jnp.int32),
            pltpu.VMEM((N,), jnp.float32),
            pltpu.VMEM((N,), jnp.float32),
            pltpu.VMEM((KPAD,), jnp.float32),
            pltpu.VMEM((KPAD,), jnp.float32),
        ],
    )


def kernel(mask_coarse_logits):
    img = mask_coarse_logits[:, :, :, 0].reshape(B, H * W)
    coords = jax.random.uniform(jax.random.key(1), (B, N, 2),
                                dtype=jnp.float32)
    unnorm = coords * jnp.array([H - 1, W - 1], dtype=jnp.float32)
    ux = unnorm[:, :, 0]
    uy = unnorm[:, :, 1]
    u_abs = _sc_interp()(img, ux, uy)
    ranks = _tc_ranks(u_abs)
    cx = coords[:, :, 0]
    cy = coords[:, :, 1]
    ox, oy = _sc_scatter()(ranks, cx, cy)
    top = jnp.stack([ox[:, :K], oy[:, :K]], axis=-1)
    extra = jax.random.uniform(jax.random.key(2), (B, N_EXTRA, 2),
                               dtype=jnp.float32)
    return jnp.concatenate([top, extra], axis=1)


# speedup vs baseline: 11.1161x; 1.3682x over previous
import functools

import jax
import jax.numpy as jnp
from jax import lax
from jax.experimental import pallas as pl
from jax.experimental.pallas import tpu as pltpu
from jax.experimental.pallas import tpu_sc as plsc

B = 512
H = 28
W = 28
N = W * H * 3
K = int(0.75 * N)
N_EXTRA = N - K
NPAD = 2432
LANES = 16
NCHUNK = N // LANES
NC, NS = 2, 16
NW = NC * NS
ROWS_PER_W = B // NW
SENTINEL = 1e30
OUT_ROW = 2 * N
TOP_LEN = 2 * K
EXTRA_LEN = 2 * N_EXTRA


@functools.cache
def _sc_mesh():
    return plsc.VectorSubcoreMesh(
        core_axis_name="c", subcore_axis_name="s",
        num_cores=NC, num_subcores=NS)


def _sc_interp_body(img_hbm, crd_hbm, out_hbm, img_v, crd_v, out_v):
    wid = lax.axis_index("s") * NC + lax.axis_index("c")
    base = wid * ROWS_PER_W
    lane = lax.iota(jnp.int32, LANES)

    def row_body(r, carry):
        b = base + r
        pltpu.sync_copy(img_hbm.at[b], img_v)
        pltpu.sync_copy(crd_hbm.at[b], crd_v)

        def chunk_body(i, c2):
            s = i * LANES
            pos = 2 * s + 2 * lane
            x = plsc.load_gather(crd_v, [pos])
            y = plsc.load_gather(crd_v, [pos + 1])
            u0 = x * jnp.float32(H - 1)
            u1 = y * jnp.float32(W - 1)
            i0 = u0.astype(jnp.int32)
            i1 = u1.astype(jnp.int32)
            f0 = i0.astype(jnp.float32)
            f1 = i1.astype(jnp.float32)
            m0 = u0 - f0
            m1 = u1 - f1
            c0 = i0 + (u0 > f0).astype(jnp.int32)
            c1 = i1 + (u1 > f1).astype(jnp.int32)
            p1 = plsc.load_gather(img_v, [i0 * W + i1])
            p2 = plsc.load_gather(img_v, [c0 * W + i1])
            p3 = plsc.load_gather(img_v, [i0 * W + c1])
            p4 = plsc.load_gather(img_v, [c0 * W + c1])
            a = p1 * (1.0 - m0) + p2 * m0
            bq = p3 * (1.0 - m0) + p4 * m0
            v = a * (1.0 - m1) + bq * m1
            out_v[pl.ds(s, LANES)] = jnp.abs(v)
            return c2

        lax.fori_loop(0, NCHUNK, chunk_body, 0)
        for t in range(NCHUNK, NPAD // LANES):
            out_v[pl.ds(t * LANES, LANES)] = jnp.full((LANES,), SENTINEL,
                                                      jnp.float32)
        pltpu.sync_copy(out_v, out_hbm.at[b])
        return carry

    lax.fori_loop(0, ROWS_PER_W, row_body, 0)


@functools.cache
def _sc_interp():
    return pl.kernel(
        _sc_interp_body,
        out_type=jax.ShapeDtypeStruct((B, NPAD), jnp.float32),
        mesh=_sc_mesh(),
        compiler_params=pltpu.CompilerParams(use_tc_tiling_on_sc=False,
                                             needs_layout_passes=False),
        scratch_types=[
            pltpu.VMEM((H * W,), jnp.float32),
            pltpu.VMEM((OUT_ROW,), jnp.float32),
            pltpu.VMEM((NPAD,), jnp.float32),
        ],
    )


def _tc_rank_body(u_ref, rank_ref, bc_ref):
    nt = NPAD // 128
    sub = lax.broadcasted_iota(jnp.int32, (128, 128), 0)
    lane = lax.broadcasted_iota(jnp.int32, (128, 128), 1)
    tri = sub < lane

    for jb in range(nt):
        uj = u_ref[0, :, pl.ds(jb * 128, 128)]
        bc_ref[pl.ds(jb * 128, 128), :] = jnp.broadcast_to(
            uj, (128, 128)).T

    for ic in range(nt):
        ui = u_ref[0, :, pl.ds(ic * 128, 128)]
        acc = jnp.zeros((128, 128), jnp.float32)
        for jb in range(nt):
            t = bc_ref[pl.ds(jb * 128, 128), :]
            if jb < ic:
                acc = jnp.where(t <= ui, acc + 1.0, acc)
            elif jb == ic:
                acc = jnp.where((t < ui) | ((t <= ui) & tri), acc + 1.0, acc)
            else:
                acc = jnp.where(t < ui, acc + 1.0, acc)
        rank = jnp.sum(acc, axis=0, keepdims=True)
        rank_ref[0, :, pl.ds(ic * 128, 128)] = rank.astype(jnp.int32)


def _tc_ranks(u):
    u3 = u.reshape(B, 1, NPAD)
    out = pl.pallas_call(
        _tc_rank_body,
        grid=(B,),
        in_specs=[pl.BlockSpec((1, 1, NPAD), lambda b: (b, 0, 0))],
        out_specs=pl.BlockSpec((1, 1, NPAD), lambda b: (b, 0, 0)),
        out_shape=jax.ShapeDtypeStruct((B, 1, NPAD), jnp.int32),
        scratch_shapes=[pltpu.VMEM((NPAD, 128), jnp.float32)],
    )(u3)
    return out.reshape(B, NPAD)


def _sc_scatter_body(rank_hbm, crd_hbm, extra_hbm, out_hbm,
                     rank_v, crd_v, out_v):
    wid = lax.axis_index("s") * NC + lax.axis_index("c")
    base = wid * ROWS_PER_W
    lane = lax.iota(jnp.int32, LANES)

    def row_body(r, carry):
        b = base + r
        pltpu.sync_copy(rank_hbm.at[b], rank_v)
        pltpu.sync_copy(crd_hbm.at[b], crd_v)
        pltpu.sync_copy(extra_hbm.at[b], out_v.at[pl.ds(TOP_LEN, EXTRA_LEN)])

        def chunk_body(i, c2):
            s = i * LANES
            pos = 2 * s + 2 * lane
            rk = rank_v[pl.ds(s, LANES)]
            msk = rk < K
            x = plsc.load_gather(crd_v, [pos])
            y = plsc.load_gather(crd_v, [pos + 1])
            plsc.store_scatter(out_v, [2 * rk], x, mask=msk)
            plsc.store_scatter(out_v, [2 * rk + 1], y, mask=msk)
            return c2

        lax.fori_loop(0, NCHUNK, chunk_body, 0)
        pltpu.sync_copy(out_v, out_hbm.at[b])
        return carry

    lax.fori_loop(0, ROWS_PER_W, row_body, 0)


@functools.cache
def _sc_scatter():
    return pl.kernel(
        _sc_scatter_body,
        out_type=jax.ShapeDtypeStruct((B, OUT_ROW), jnp.float32),
        mesh=_sc_mesh(),
        compiler_params=pltpu.CompilerParams(use_tc_tiling_on_sc=False,
                                             needs_layout_passes=False),
        scratch_types=[
            pltpu.VMEM((NPAD,), jnp.int32),
            pltpu.VMEM((OUT_ROW,), jnp.float32),
            pltpu.VMEM((OUT_ROW,), jnp.float32),
        ],
    )


def kernel(mask_coarse_logits):
    img = mask_coarse_logits[:, :, :, 0].reshape(B, H * W)
    coords = jax.random.uniform(jax.random.key(1), (B, N, 2),
                                dtype=jnp.float32).reshape(B, OUT_ROW)
    extra = jax.random.uniform(jax.random.key(2), (B, N_EXTRA, 2),
                               dtype=jnp.float32).reshape(B, EXTRA_LEN)
    u_abs = _sc_interp()(img, coords)
    ranks = _tc_ranks(u_abs)
    out = _sc_scatter()(ranks, coords, extra)
    return out.reshape(B, N, 2)
